# baseline (speedup 1.0000x reference)
import functools

import jax
import jax.numpy as jnp
from jax import lax
from jax.experimental import pallas as pl
from jax.experimental.pallas import tpu as pltpu
from jax.experimental.pallas import tpu_sc as plsc

EMB = 32
BATCH = 16384


def _make_kernel(vocab, emb, batch):
    info = plsc.get_sparse_core_info()
    nw = info.num_cores * info.num_subcores
    b_per_w = batch // nw
    mesh = plsc.VectorSubcoreMesh(core_axis_name="c", subcore_axis_name="s")

    @functools.partial(
        pl.kernel,
        mesh=mesh,
        out_type=[
            jax.ShapeDtypeStruct((batch, emb), jnp.float32),
            jax.ShapeDtypeStruct((batch, emb), jnp.float32),
        ],
        scratch_types=[
            pltpu.VMEM((b_per_w,), jnp.int32),
            pltpu.VMEM((b_per_w,), jnp.int32),
            pltpu.VMEM((b_per_w, emb), jnp.float32),
            pltpu.VMEM((b_per_w, emb), jnp.float32),
            pltpu.SemaphoreType.DMA,
            pltpu.SemaphoreType.DMA,
        ],
        compiler_params=pltpu.CompilerParams(use_tc_tiling_on_sc=False),
    )
    def gather_kernel(table_hbm, uidx_hbm, iidx_hbm, out_u, out_i,
                      uidx_v, iidx_v, urows_v, irows_v, sem_u, sem_i):
        wid = lax.axis_index("s") * info.num_cores + lax.axis_index("c")
        base = wid * b_per_w
        pltpu.sync_copy(uidx_hbm.at[pl.ds(base, b_per_w)], uidx_v)
        pltpu.sync_copy(iidx_hbm.at[pl.ds(base, b_per_w)], iidx_v)
        cu = pltpu.async_copy(table_hbm.at[uidx_v], urows_v, sem_u)
        ci = pltpu.async_copy(table_hbm.at[iidx_v], irows_v, sem_i)
        cu.wait()
        pltpu.sync_copy(urows_v, out_u.at[pl.ds(base, b_per_w)])
        ci.wait()
        pltpu.sync_copy(irows_v, out_i.at[pl.ds(base, b_per_w)])

    return gather_kernel


def kernel(embeddings, user_ids, item_ids):
    vocab, emb = embeddings.shape
    fn = _make_kernel(vocab, emb, user_ids.shape[0])
    users_emb, items_emb = fn(embeddings, user_ids, item_ids)
    return (users_emb, items_emb)
